# Initial kernel scaffold; baseline (speedup 1.0000x reference)
import jax
import jax.numpy as jnp
from jax.experimental import pallas as pl


def kernel(xCellCenters, xFace, cf_ei, cf_ea, fp_ei, fp_ea, pp0_ei, pp0_ea, pp1_ei, pp1_ea, pp2_ei, pp2_ea, pp3_ei, pp3_ea, pp4_ei, pp4_ea, pool1_ei, pool1_ea, pool2_ei, pool2_ea, pool3_ei, pool3_ea, pool4_ei, pool4_ea, pc_ei, pc_ea, params):
    raise NotImplementedError("write your pallas kernel here")


# speedup vs baseline: 1.8962x; 1.8962x over previous
import functools

import jax
import jax.numpy as jnp
from jax import lax
from jax.experimental import pallas as pl
from jax.experimental.pallas import tpu as pltpu
from jax.experimental.pallas import tpu_sc as plsc

F32 = jnp.float32
I32 = jnp.int32

NODE_PAD = 4096
BR = 2048
LANES = 16
SPMEM_BUDGET = 5_500_000


def _rup(x, m):
    return -(-x // m) * m


def _mm(x, w, a=None, relu=False):
    np_, k = x.shape
    dout = w.shape[1]
    grid = np_ // BR

    def body(x_ref, w_ref, *rest):
        if a is not None:
            a_ref, o_ref = rest
        else:
            (o_ref,) = rest
        acc = jnp.dot(x_ref[...], w_ref[...], preferred_element_type=F32)
        if a is not None:
            acc = acc + a_ref[...]
        if relu:
            acc = jnp.maximum(acc, 0.0)
        o_ref[...] = acc

    in_specs = [
        pl.BlockSpec((BR, k), lambda i: (i, 0)),
        pl.BlockSpec((k, dout), lambda i: (0, 0)),
    ]
    ins = [x, w]
    if a is not None:
        in_specs.append(pl.BlockSpec((BR, dout), lambda i: (i, 0)))
        ins.append(a)
    return pl.pallas_call(
        body,
        grid=(grid,),
        in_specs=in_specs,
        out_specs=pl.BlockSpec((BR, dout), lambda i: (i, 0)),
        out_shape=jax.ShapeDtypeStruct((np_, dout), F32),
    )(*ins)


def _inorm(x, n_true):
    np_, d = x.shape
    grid = np_ // BR

    def stats(x_ref, m_ref, s_ref):
        i = pl.program_id(0)
        xb = x_ref[...]
        rid = lax.broadcasted_iota(I32, (BR, 1), 0) + i * BR
        xb = jnp.where(rid < n_true, xb, 0.0)
        s = jnp.sum(xb, axis=0, keepdims=True)
        q = jnp.sum(xb * xb, axis=0, keepdims=True)

        @pl.when(i == 0)
        def _():
            m_ref[...] = s
            s_ref[...] = q

        @pl.when(i > 0)
        def _():
            m_ref[...] += s
            s_ref[...] += q

        @pl.when(i == grid - 1)
        def _():
            mu = m_ref[...] * (1.0 / n_true)
            var = s_ref[...] * (1.0 / n_true) - mu * mu
            m_ref[...] = mu
            s_ref[...] = lax.rsqrt(var + 1e-5)

    m, istd = pl.pallas_call(
        stats,
        grid=(grid,),
        in_specs=[pl.BlockSpec((BR, d), lambda i: (i, 0))],
        out_specs=[pl.BlockSpec((1, d), lambda i: (0, 0))] * 2,
        out_shape=[jax.ShapeDtypeStruct((1, d), F32)] * 2,
    )(x)

    def apply(x_ref, m_ref, s_ref, o_ref):
        o_ref[...] = (x_ref[...] - m_ref[...]) * s_ref[...]

    return pl.pallas_call(
        apply,
        grid=(grid,),
        in_specs=[
            pl.BlockSpec((BR, d), lambda i: (i, 0)),
            pl.BlockSpec((1, d), lambda i: (0, 0)),
            pl.BlockSpec((1, d), lambda i: (0, 0)),
        ],
        out_specs=pl.BlockSpec((BR, d), lambda i: (i, 0)),
        out_shape=jax.ShapeDtypeStruct((np_, d), F32),
    )(x, m, istd)


def _sc_segsum(y, src, dst, ea, n_out_p, relu=False):
    np_src, d = y.shape
    r = d // LANES
    e = src.shape[0]

    ept0 = -(-e // 16)
    cb = 1024
    nch = -(-ept0 // cb)
    ept = cb * nch
    e_pad = 16 * ept
    src = jnp.pad(src, (0, e_pad - e))
    dst = jnp.pad(dst, (0, e_pad - e), constant_values=jnp.int32(2**30))
    ea = jnp.pad(ea, (0, e_pad - e))
    src2 = src.reshape(-1, 128)
    ng = cb // 128

    half = n_out_p // 2
    s_rows = max(2048, (SPMEM_BUDGET // (d * 4)) // 2048 * 2048)
    nck = -(-half // s_rows)
    s_sz = _rup(-(-half // nck), 2048)
    chunks = []
    off = 0
    while off < half:
        chunks.append((off, min(s_sz, half - off)))
        off += s_sz

    mesh = plsc.VectorSubcoreMesh(
        core_axis_name="c", subcore_axis_name="s", num_cores=2, num_subcores=16
    )

    @functools.partial(
        pl.kernel,
        mesh=mesh,
        compiler_params=pltpu.CompilerParams(use_tc_tiling_on_sc=False),
        out_type=jax.ShapeDtypeStruct((n_out_p, d), F32),
        scratch_types=[
            pltpu.VMEM((ng, 128), I32),
            pltpu.VMEM((ng, 128), I32),
            pltpu.VMEM((cb,), I32),
            pltpu.VMEM((cb,), F32),
            pltpu.VMEM((cb, d), F32),
            pltpu.VMEM_SHARED((s_sz + 8, d), F32),
            pltpu.SemaphoreType.DMA,
        ],
    )
    def k(y_h, src2_h, dst_h, ea_h, out_h,
          gidx_v, sidx_v, dst_v, ea_v, rows_v, acc_sh, sem):
        ci = lax.axis_index("c")
        si = lax.axis_index("s")
        tile_e0 = si * ept

        for c_off, c_rows in chunks:
            base = ci * half + c_off
            zrpt = c_rows // 16

            def zbuf(i, _):
                for rr in range(r):
                    rows_v[i, pl.ds(rr * 16, 16)] = jnp.zeros((16,), F32)
                return 0

            lax.fori_loop(0, cb, zbuf, 0)

            def zacc(i, _, _si=si, _zr=zrpt):
                o = pl.multiple_of(_si * _zr + i * 128, 128)
                pltpu.sync_copy(
                    rows_v.at[pl.ds(0, 128)],
                    acc_sh.at[pl.ds(o, 128)],
                )
                return 0

            lax.fori_loop(0, zrpt // 128, zacc, 0)
            plsc.subcore_barrier()

            def edge_chunk(n, _, _base=base, _c_rows=c_rows):
                e0 = pl.multiple_of(tile_e0 + n * cb, cb)
                g0 = pl.multiple_of(e0 // 128, 8)
                pltpu.sync_copy(src2_h.at[pl.ds(g0, ng)], gidx_v)
                pltpu.sync_copy(dst_h.at[pl.ds(e0, cb)], dst_v)
                pltpu.sync_copy(ea_h.at[pl.ds(e0, cb)], ea_v)
                handles = [
                    pltpu.async_copy(
                        y_h.at[gidx_v.at[g]],
                        rows_v.at[pl.ds(g * 128, 128)],
                        sem,
                    )
                    for g in range(ng)
                ]
                for h in handles:
                    h.wait()
                for j in range(cb // 16):
                    dv = dst_v[pl.ds(j * 16, 16)]
                    okm = (dv >= _base) & (dv < _base + _c_rows)
                    loc = jnp.where(okm, dv - _base, jnp.int32(s_sz))
                    sidx_v[j // 8, pl.ds((j % 8) * 16, 16)] = loc
                def scale(gg, _):
                    e16 = ea_v[pl.ds(gg * 16, 16)]
                    for l in range(16):
                        eav = lax.broadcast(e16[l], (16,))
                        ei = gg * 16 + l
                        for rr in range(r):
                            rows_v[ei, pl.ds(rr * 16, 16)] = (
                                rows_v[ei, pl.ds(rr * 16, 16)] * eav
                            )
                    return 0

                lax.fori_loop(0, cb // 16, scale, 0)
                for g in range(ng):
                    pltpu.sync_copy(
                        rows_v.at[pl.ds(g * 128, 128)],
                        acc_sh.at[sidx_v.at[g]],
                        add=True,
                    )
                return 0

            lax.fori_loop(0, nch, edge_chunk, 0)
            plsc.subcore_barrier()

            def wback(i, _, _si=si, _zr=zrpt, _base=base):
                o = pl.multiple_of(_si * _zr + i * 128, 128)
                pltpu.sync_copy(
                    acc_sh.at[pl.ds(o, 128)],
                    rows_v.at[pl.ds(0, 128)],
                )
                if relu:
                    def vmax(jj, _):
                        for rr in range(r):
                            rows_v[jj, pl.ds(rr * 16, 16)] = jnp.maximum(
                                rows_v[jj, pl.ds(rr * 16, 16)], 0.0
                            )
                        return 0

                    lax.fori_loop(0, 128, vmax, 0)
                oo = pl.multiple_of(_base + _si * _zr + i * 128, 128)
                pltpu.sync_copy(
                    rows_v.at[pl.ds(0, 128)],
                    out_h.at[pl.ds(oo, 128)],
                )
                return 0

            lax.fori_loop(0, zrpt // 128, wback, 0)
            plsc.subcore_barrier()

    return k(y, src2, dst, ea)


def _pad_rows_cols(x, np_, d):
    return jnp.pad(x, ((0, np_ - x.shape[0]), (0, d - x.shape[1])))


def _pad_w(w, k, d):
    return jnp.pad(w, ((0, k - w.shape[0]), (0, d - w.shape[1])))


def _pp_block(x, ei, ea, ws, wn, k, dout, n_true):
    np_ = x.shape[0]
    yn = _mm(x, _pad_w(wn, k, dout))
    agg = _sc_segsum(yn, ei[0], ei[1], ea[:, 0], np_)
    return _mm(x, _pad_w(ws, k, dout), a=agg, relu=True)


def kernel(xCellCenters, xFace, cf_ei, cf_ea, fp_ei, fp_ea,
           pp0_ei, pp0_ea, pp1_ei, pp1_ea, pp2_ei, pp2_ea,
           pp3_ei, pp3_ea, pp4_ei, pp4_ea,
           pool1_ei, pool1_ea, pool2_ei, pool2_ea,
           pool3_ei, pool3_ea, pool4_ei, pool4_ea,
           pc_ei, pc_ea, params):
    p = params
    n_c = xCellCenters.shape[1]
    n_f = xFace.shape[1]
    n0 = pool1_ei.shape[1]
    n1 = pool2_ei.shape[1]
    n2 = pool3_ei.shape[1]
    n3 = pool4_ei.shape[1]
    n4 = 400
    ncp = _rup(n_c, NODE_PAD)
    nfp = _rup(n_f, NODE_PAD)
    n0p = _rup(n0, NODE_PAD)
    n1p = _rup(n1, NODE_PAD)
    n2p = _rup(n2, NODE_PAD)
    n3p = _rup(n3, NODE_PAD)
    n4p = _rup(n4, NODE_PAD)

    xcn = _inorm(_pad_rows_cols(xCellCenters[0], ncp, 8), n_c)
    xfn = _inorm(_pad_rows_cols(xFace[0], nfp, 8), n_f)

    y = _mm(xcn, _pad_w(p["W_cf"], 8, 32))
    h_agg = _sc_segsum(y, cf_ei[0], cf_ei[1], cf_ea[:, 0], nfp, relu=True)
    h = jnp.pad(
        jnp.concatenate([h_agg[:, :20], xfn[:, :4]], axis=1),
        ((0, 0), (0, 8)),
    )

    y = _mm(h, _pad_w(p["W_fp"], 32, 32))
    c1 = _sc_segsum(y, fp_ei[0], fp_ei[1], fp_ea[:, 0], n0p, relu=True)
    c1n = _inorm(c1, n0)

    x = _sc_segsum(c1n, pool1_ei[0], pool1_ei[1], pool1_ea[:, 0], n1p)
    c2n = _inorm(_pp_block(x, pp1_ei, pp1_ea, p["Ws2"], p["Wn2"], 32, 32, n1), n1)
    x = _sc_segsum(c2n, pool2_ei[0], pool2_ei[1], pool2_ea[:, 0], n2p)
    c3n = _inorm(_pp_block(x, pp2_ei, pp2_ea, p["Ws3"], p["Wn3"], 32, 32, n2), n2)
    x = _sc_segsum(c3n, pool3_ei[0], pool3_ei[1], pool3_ea[:, 0], n3p)
    c4n = _inorm(_pp_block(x, pp3_ei, pp3_ea, p["Ws4"], p["Wn4"], 32, 48, n3), n3)
    x = _sc_segsum(c4n, pool4_ei[0], pool4_ei[1], pool4_ea[:, 0], n4p)

    x = _pp_block(x, pp4_ei, pp4_ea, p["Ws5a"], p["Wn5a"], 48, 48, n4)
    x = _pp_block(x, pp4_ei, pp4_ea, p["Ws5b"], p["Wn5b"], 48, 48, n4)

    u = _sc_segsum(x, pool4_ei[1], pool4_ei[0], pool4_ea[:, 0], n3p)
    c5n = _inorm(u, n3)
    x = jnp.concatenate([c5n[:, :48], c4n[:, :48]], axis=1)
    x = _pp_block(x, pp3_ei, pp3_ea, p["Ws6"], p["Wn6"], 96, 48, n3)

    u = _sc_segsum(x, pool3_ei[1], pool3_ei[0], pool3_ea[:, 0], n2p)
    c6n = _inorm(u, n2)
    x = jnp.pad(
        jnp.concatenate([c6n[:, :48], c3n[:, :24]], axis=1), ((0, 0), (0, 8))
    )
    x = _pp_block(x, pp2_ei, pp2_ea, p["Ws7"], p["Wn7"], 80, 48, n2)

    u = _sc_segsum(x, pool2_ei[1], pool2_ei[0], pool2_ea[:, 0], n1p)
    c7n = _inorm(u, n1)
    x = jnp.pad(
        jnp.concatenate([c7n[:, :48], c2n[:, :24]], axis=1), ((0, 0), (0, 8))
    )
    x = _pp_block(x, pp1_ei, pp1_ea, p["Ws8"], p["Wn8"], 80, 32, n1)

    u = _sc_segsum(x, pool1_ei[1], pool1_ei[0], pool1_ea[:, 0], n0p)
    c8n = _inorm(u, n0)
    x = jnp.concatenate([c8n[:, :24], c1n[:, :24]], axis=1)
    x = _pp_block(x, pp0_ei, pp0_ea, p["Ws9"], p["Wn9"], 48, 32, n0)

    y = _mm(x, _pad_w(p["W_pc"], 32, 32))
    c9 = _sc_segsum(y, pc_ei[0], pc_ei[1], pc_ea[:, 0], ncp, relu=True)
    c9n = _inorm(c9, n_c)
    return c9n[:n_c, :24][None]


# speedup vs baseline: 1.9026x; 1.0034x over previous
import functools

import jax
import jax.numpy as jnp
from jax import lax
from jax.experimental import pallas as pl
from jax.experimental.pallas import tpu as pltpu
from jax.experimental.pallas import tpu_sc as plsc

F32 = jnp.float32
I32 = jnp.int32

NODE_PAD = 4096
BR = 2048
LANES = 16
SPMEM_BUDGET = 5_500_000


def _rup(x, m):
    return -(-x // m) * m


def _mm(x, w, a=None, relu=False):
    np_, k = x.shape
    dout = w.shape[1]
    grid = np_ // BR

    def body(x_ref, w_ref, *rest):
        if a is not None:
            a_ref, o_ref = rest
        else:
            (o_ref,) = rest
        acc = jnp.dot(x_ref[...], w_ref[...], preferred_element_type=F32)
        if a is not None:
            acc = acc + a_ref[...]
        if relu:
            acc = jnp.maximum(acc, 0.0)
        o_ref[...] = acc

    in_specs = [
        pl.BlockSpec((BR, k), lambda i: (i, 0)),
        pl.BlockSpec((k, dout), lambda i: (0, 0)),
    ]
    ins = [x, w]
    if a is not None:
        in_specs.append(pl.BlockSpec((BR, dout), lambda i: (i, 0)))
        ins.append(a)
    return pl.pallas_call(
        body,
        grid=(grid,),
        in_specs=in_specs,
        out_specs=pl.BlockSpec((BR, dout), lambda i: (i, 0)),
        out_shape=jax.ShapeDtypeStruct((np_, dout), F32),
    )(*ins)


def _inorm(x, n_true):
    np_, d = x.shape
    grid = np_ // BR

    def stats(x_ref, m_ref, s_ref):
        i = pl.program_id(0)
        xb = x_ref[...]
        rid = lax.broadcasted_iota(I32, (BR, 1), 0) + i * BR
        xb = jnp.where(rid < n_true, xb, 0.0)
        s = jnp.sum(xb, axis=0, keepdims=True)
        q = jnp.sum(xb * xb, axis=0, keepdims=True)

        @pl.when(i == 0)
        def _():
            m_ref[...] = s
            s_ref[...] = q

        @pl.when(i > 0)
        def _():
            m_ref[...] += s
            s_ref[...] += q

        @pl.when(i == grid - 1)
        def _():
            mu = m_ref[...] * (1.0 / n_true)
            var = s_ref[...] * (1.0 / n_true) - mu * mu
            m_ref[...] = mu
            s_ref[...] = lax.rsqrt(var + 1e-5)

    m, istd = pl.pallas_call(
        stats,
        grid=(grid,),
        in_specs=[pl.BlockSpec((BR, d), lambda i: (i, 0))],
        out_specs=[pl.BlockSpec((1, d), lambda i: (0, 0))] * 2,
        out_shape=[jax.ShapeDtypeStruct((1, d), F32)] * 2,
    )(x)

    def apply(x_ref, m_ref, s_ref, o_ref):
        o_ref[...] = (x_ref[...] - m_ref[...]) * s_ref[...]

    return pl.pallas_call(
        apply,
        grid=(grid,),
        in_specs=[
            pl.BlockSpec((BR, d), lambda i: (i, 0)),
            pl.BlockSpec((1, d), lambda i: (0, 0)),
            pl.BlockSpec((1, d), lambda i: (0, 0)),
        ],
        out_specs=pl.BlockSpec((BR, d), lambda i: (i, 0)),
        out_shape=jax.ShapeDtypeStruct((np_, d), F32),
    )(x, m, istd)


def _sc_segsum(y, src, dst, ea, n_out_p, relu=False):
    np_src, d = y.shape
    r = d // LANES
    e = src.shape[0]

    ept0 = -(-e // 16)
    cb = 1024
    nch = -(-ept0 // cb)
    ept = cb * nch
    e_pad = 16 * ept
    src = jnp.pad(src, (0, e_pad - e))
    dst = jnp.pad(dst, (0, e_pad - e), constant_values=jnp.int32(2**30))
    ea = jnp.pad(ea, (0, e_pad - e))
    src2 = src.reshape(-1, 128)
    ng = cb // 128

    half = n_out_p // 2
    s_rows = max(2048, (SPMEM_BUDGET // (d * 4)) // 2048 * 2048)
    nck = -(-half // s_rows)
    s_sz = _rup(-(-half // nck), 2048)
    chunks = []
    off = 0
    while off < half:
        chunks.append((off, min(s_sz, half - off)))
        off += s_sz

    mesh = plsc.VectorSubcoreMesh(
        core_axis_name="c", subcore_axis_name="s", num_cores=2, num_subcores=16
    )

    @functools.partial(
        pl.kernel,
        mesh=mesh,
        compiler_params=pltpu.CompilerParams(use_tc_tiling_on_sc=False),
        out_type=jax.ShapeDtypeStruct((n_out_p, d), F32),
        scratch_types=[
            pltpu.VMEM((ng, 128), I32),
            pltpu.VMEM((ng, 128), I32),
            pltpu.VMEM((cb,), I32),
            pltpu.VMEM((cb,), F32),
            pltpu.VMEM((cb, d), F32),
            pltpu.VMEM_SHARED((s_sz + 8, d), F32),
            pltpu.SemaphoreType.DMA,
        ],
    )
    def k(y_h, src2_h, dst_h, ea_h, out_h,
          gidx_v, sidx_v, dst_v, ea_v, rows_v, acc_sh, sem):
        ci = lax.axis_index("c")
        si = lax.axis_index("s")
        tile_e0 = si * ept

        for c_off, c_rows in chunks:
            base = ci * half + c_off
            zrpt = c_rows // 16

            def zbuf(i, _):
                for rr in range(r):
                    rows_v[i, pl.ds(rr * 16, 16)] = jnp.zeros((16,), F32)
                return 0

            lax.fori_loop(0, cb, zbuf, 0)

            def zacc(i, _, _si=si, _zr=zrpt):
                o = pl.multiple_of(_si * _zr + i * 128, 128)
                pltpu.sync_copy(
                    rows_v.at[pl.ds(0, 128)],
                    acc_sh.at[pl.ds(o, 128)],
                )
                return 0

            lax.fori_loop(0, zrpt // 128, zacc, 0)
            plsc.subcore_barrier()

            def edge_chunk(n, _, _base=base, _c_rows=c_rows):
                e0 = pl.multiple_of(tile_e0 + n * cb, cb)
                g0 = pl.multiple_of(e0 // 128, 8)
                loads = [
                    pltpu.async_copy(src2_h.at[pl.ds(g0, ng)], gidx_v, sem),
                    pltpu.async_copy(dst_h.at[pl.ds(e0, cb)], dst_v, sem),
                    pltpu.async_copy(ea_h.at[pl.ds(e0, cb)], ea_v, sem),
                ]
                for h in loads:
                    h.wait()
                handles = [
                    pltpu.async_copy(
                        y_h.at[gidx_v.at[g]],
                        rows_v.at[pl.ds(g * 128, 128)],
                        sem,
                    )
                    for g in range(ng)
                ]
                for h in handles:
                    h.wait()
                for j in range(cb // 16):
                    dv = dst_v[pl.ds(j * 16, 16)]
                    okm = (dv >= _base) & (dv < _base + _c_rows)
                    loc = jnp.where(okm, dv - _base, jnp.int32(s_sz))
                    sidx_v[j // 8, pl.ds((j % 8) * 16, 16)] = loc
                def scale(gg, _):
                    e16 = ea_v[pl.ds(gg * 16, 16)]
                    for l in range(16):
                        eav = lax.broadcast(e16[l], (16,))
                        ei = gg * 16 + l
                        for rr in range(r):
                            rows_v[ei, pl.ds(rr * 16, 16)] = (
                                rows_v[ei, pl.ds(rr * 16, 16)] * eav
                            )
                    return 0

                lax.fori_loop(0, cb // 16, scale, 0)
                scats = [
                    pltpu.async_copy(
                        rows_v.at[pl.ds(g * 128, 128)],
                        acc_sh.at[sidx_v.at[g]],
                        sem,
                        add=True,
                    )
                    for g in range(ng)
                ]
                for h in scats:
                    h.wait()
                return 0

            lax.fori_loop(0, nch, edge_chunk, 0)
            plsc.subcore_barrier()

            def wback(i, _, _si=si, _zr=zrpt, _base=base):
                o = pl.multiple_of(_si * _zr + i * 128, 128)
                pltpu.sync_copy(
                    acc_sh.at[pl.ds(o, 128)],
                    rows_v.at[pl.ds(0, 128)],
                )
                if relu:
                    def vmax(jj, _):
                        for rr in range(r):
                            rows_v[jj, pl.ds(rr * 16, 16)] = jnp.maximum(
                                rows_v[jj, pl.ds(rr * 16, 16)], 0.0
                            )
                        return 0

                    lax.fori_loop(0, 128, vmax, 0)
                oo = pl.multiple_of(_base + _si * _zr + i * 128, 128)
                pltpu.sync_copy(
                    rows_v.at[pl.ds(0, 128)],
                    out_h.at[pl.ds(oo, 128)],
                )
                return 0

            lax.fori_loop(0, zrpt // 128, wback, 0)
            plsc.subcore_barrier()

    return k(y, src2, dst, ea)


def _pad_rows_cols(x, np_, d):
    return jnp.pad(x, ((0, np_ - x.shape[0]), (0, d - x.shape[1])))


def _pad_w(w, k, d):
    return jnp.pad(w, ((0, k - w.shape[0]), (0, d - w.shape[1])))


def _pp_block(x, ei, ea, ws, wn, k, dout, n_true):
    np_ = x.shape[0]
    yn = _mm(x, _pad_w(wn, k, dout))
    agg = _sc_segsum(yn, ei[0], ei[1], ea[:, 0], np_)
    return _mm(x, _pad_w(ws, k, dout), a=agg, relu=True)


def kernel(xCellCenters, xFace, cf_ei, cf_ea, fp_ei, fp_ea,
           pp0_ei, pp0_ea, pp1_ei, pp1_ea, pp2_ei, pp2_ea,
           pp3_ei, pp3_ea, pp4_ei, pp4_ea,
           pool1_ei, pool1_ea, pool2_ei, pool2_ea,
           pool3_ei, pool3_ea, pool4_ei, pool4_ea,
           pc_ei, pc_ea, params):
    p = params
    n_c = xCellCenters.shape[1]
    n_f = xFace.shape[1]
    n0 = pool1_ei.shape[1]
    n1 = pool2_ei.shape[1]
    n2 = pool3_ei.shape[1]
    n3 = pool4_ei.shape[1]
    n4 = 400
    ncp = _rup(n_c, NODE_PAD)
    nfp = _rup(n_f, NODE_PAD)
    n0p = _rup(n0, NODE_PAD)
    n1p = _rup(n1, NODE_PAD)
    n2p = _rup(n2, NODE_PAD)
    n3p = _rup(n3, NODE_PAD)
    n4p = _rup(n4, NODE_PAD)

    xcn = _inorm(_pad_rows_cols(xCellCenters[0], ncp, 8), n_c)
    xfn = _inorm(_pad_rows_cols(xFace[0], nfp, 8), n_f)

    y = _mm(xcn, _pad_w(p["W_cf"], 8, 32))
    h_agg = _sc_segsum(y, cf_ei[0], cf_ei[1], cf_ea[:, 0], nfp, relu=True)
    h = jnp.pad(
        jnp.concatenate([h_agg[:, :20], xfn[:, :4]], axis=1),
        ((0, 0), (0, 8)),
    )

    y = _mm(h, _pad_w(p["W_fp"], 32, 32))
    c1 = _sc_segsum(y, fp_ei[0], fp_ei[1], fp_ea[:, 0], n0p, relu=True)
    c1n = _inorm(c1, n0)

    x = _sc_segsum(c1n, pool1_ei[0], pool1_ei[1], pool1_ea[:, 0], n1p)
    c2n = _inorm(_pp_block(x, pp1_ei, pp1_ea, p["Ws2"], p["Wn2"], 32, 32, n1), n1)
    x = _sc_segsum(c2n, pool2_ei[0], pool2_ei[1], pool2_ea[:, 0], n2p)
    c3n = _inorm(_pp_block(x, pp2_ei, pp2_ea, p["Ws3"], p["Wn3"], 32, 32, n2), n2)
    x = _sc_segsum(c3n, pool3_ei[0], pool3_ei[1], pool3_ea[:, 0], n3p)
    c4n = _inorm(_pp_block(x, pp3_ei, pp3_ea, p["Ws4"], p["Wn4"], 32, 48, n3), n3)
    x = _sc_segsum(c4n, pool4_ei[0], pool4_ei[1], pool4_ea[:, 0], n4p)

    x = _pp_block(x, pp4_ei, pp4_ea, p["Ws5a"], p["Wn5a"], 48, 48, n4)
    x = _pp_block(x, pp4_ei, pp4_ea, p["Ws5b"], p["Wn5b"], 48, 48, n4)

    u = _sc_segsum(x, pool4_ei[1], pool4_ei[0], pool4_ea[:, 0], n3p)
    c5n = _inorm(u, n3)
    x = jnp.concatenate([c5n[:, :48], c4n[:, :48]], axis=1)
    x = _pp_block(x, pp3_ei, pp3_ea, p["Ws6"], p["Wn6"], 96, 48, n3)

    u = _sc_segsum(x, pool3_ei[1], pool3_ei[0], pool3_ea[:, 0], n2p)
    c6n = _inorm(u, n2)
    x = jnp.pad(
        jnp.concatenate([c6n[:, :48], c3n[:, :24]], axis=1), ((0, 0), (0, 8))
    )
    x = _pp_block(x, pp2_ei, pp2_ea, p["Ws7"], p["Wn7"], 80, 48, n2)

    u = _sc_segsum(x, pool2_ei[1], pool2_ei[0], pool2_ea[:, 0], n1p)
    c7n = _inorm(u, n1)
    x = jnp.pad(
        jnp.concatenate([c7n[:, :48], c2n[:, :24]], axis=1), ((0, 0), (0, 8))
    )
    x = _pp_block(x, pp1_ei, pp1_ea, p["Ws8"], p["Wn8"], 80, 32, n1)

    u = _sc_segsum(x, pool1_ei[1], pool1_ei[0], pool1_ea[:, 0], n0p)
    c8n = _inorm(u, n0)
    x = jnp.concatenate([c8n[:, :24], c1n[:, :24]], axis=1)
    x = _pp_block(x, pp0_ei, pp0_ea, p["Ws9"], p["Wn9"], 48, 32, n0)

    y = _mm(x, _pad_w(p["W_pc"], 32, 32))
    c9 = _sc_segsum(y, pc_ei[0], pc_ei[1], pc_ea[:, 0], ncp, relu=True)
    c9n = _inorm(c9, n_c)
    return c9n[:n_c, :24][None]
